# Initial kernel scaffold; baseline (speedup 1.0000x reference)
import jax
import jax.numpy as jnp
from jax.experimental import pallas as pl


def kernel(inputs, weights):
    raise NotImplementedError("write your pallas kernel here")


# speedup vs baseline: 2.7557x; 2.7557x over previous
import functools

import jax
import jax.numpy as jnp
from jax import lax
from jax.experimental import pallas as pl
from jax.experimental.pallas import tpu as pltpu
from jax.experimental.pallas import tpu_sc as plsc

B, L, D = 16384, 50, 32
NC, NS = 2, 16
NW = NC * NS
BAGS_PER_W = B // NW
T = 32
N_IT = BAGS_PER_W // T
IDX_CHUNK = T * L
INV_L = 1.0 / L


def _body(idx_hbm, w_hbm, out_hbm, idx_v, rows_v, out_v, sem):
    wid = lax.axis_index("s") * NC + lax.axis_index("c")
    base_bag = wid * BAGS_PER_W

    @pl.loop(0, N_IT)
    def _tile(t):
        bag0 = base_bag + t * T
        pltpu.sync_copy(idx_hbm.at[pl.ds(bag0 * L, IDX_CHUNK)], idx_v)
        pltpu.async_copy(w_hbm.at[idx_v], rows_v, sem).wait()

        @pl.loop(0, T)
        def _bag(b):
            r0 = b * L
            acc0 = jnp.zeros((16,), jnp.float32)
            acc1 = jnp.zeros((16,), jnp.float32)
            for j in range(L):
                acc0 = acc0 + rows_v[r0 + j, pl.ds(0, 16)]
                acc1 = acc1 + rows_v[r0 + j, pl.ds(16, 16)]
            out_v[b, pl.ds(0, 16)] = acc0 * INV_L
            out_v[b, pl.ds(16, 16)] = acc1 * INV_L

        pltpu.sync_copy(out_v, out_hbm.at[pl.ds(bag0, T), :])


@jax.jit
def kernel(inputs, weights):
    flat_idx = inputs.reshape(-1)
    mesh = plsc.VectorSubcoreMesh(
        core_axis_name="c", subcore_axis_name="s",
        num_cores=NC, num_subcores=NS)
    k = pl.kernel(
        _body,
        out_type=jax.ShapeDtypeStruct((B, D), jnp.float32),
        mesh=mesh,
        scratch_types=[
            pltpu.VMEM((IDX_CHUNK,), jnp.int32),
            pltpu.VMEM((IDX_CHUNK, D), jnp.float32),
            pltpu.VMEM((T, D), jnp.float32),
            pltpu.SemaphoreType.DMA,
        ],
        compiler_params=pltpu.CompilerParams(use_tc_tiling_on_sc=False),
    )
    return k(flat_idx, weights)
